# Initial kernel scaffold; baseline (speedup 1.0000x reference)
import jax
import jax.numpy as jnp
from jax.experimental import pallas as pl


def kernel(state, assigner_logits, agent_emb, W1, b1, W2, b2, W3, b3):
    raise NotImplementedError("write your pallas kernel here")


# speedup vs baseline: 1.2841x; 1.2841x over previous
import jax
import jax.numpy as jnp
from jax.experimental import pallas as pl
from jax.experimental.pallas import tpu as pltpu

_B, _G, _E = 4, 64, 8
_DS, _DE, _H, _A = 1024, 64, 1024, 16
_DIN = _DS + _DE
_N = _B * _G


def _mlp_kernel(eidx_ref, state_ref, emb_ref, w1_ref, b1_ref, w2_ref, b2_ref,
                w3_ref, b3_ref, out_ref):
    e = pl.program_id(0)
    bf = jnp.bfloat16
    w1 = w1_ref[0]
    sp = jnp.dot(state_ref[...].astype(bf), w1[_DE:, :].astype(bf),
                 preferred_element_type=jnp.float32)
    ep = jnp.dot(emb_ref[...].astype(bf), w1[:_DE, :].astype(bf),
                 preferred_element_type=jnp.float32)
    h1 = jnp.maximum(sp[:, None, :] + ep[None, :, :] + b1_ref[...],
                     0.0)
    h1b = h1.reshape(_N, _H).astype(bf)
    h2 = jnp.maximum(
        jnp.dot(h1b, w2_ref[0].astype(bf), preferred_element_type=jnp.float32)
        + b2_ref[0], 0.0)
    o = (jnp.dot(h2.astype(bf), w3_ref[0].astype(bf),
                 preferred_element_type=jnp.float32) + b3_ref[0])
    contrib = jnp.where(eidx_ref[...] == e, o, 0.0)

    @pl.when(e == 0)
    def _():
        out_ref[...] = contrib

    @pl.when(e != 0)
    def _():
        out_ref[...] = out_ref[...] + contrib


def _run_mlp(eidx, state, agent_emb, W1, b1, W2, b2, W3, b3):
    return pl.pallas_call(
        _mlp_kernel,
        grid=(_E,),
        in_specs=[
            pl.BlockSpec((_N, 1), lambda e: (0, 0)),
            pl.BlockSpec((_B, _DS), lambda e: (0, 0)),
            pl.BlockSpec((_G, _DE), lambda e: (0, 0)),
            pl.BlockSpec((1, _DIN, _H), lambda e: (e, 0, 0)),
            pl.BlockSpec((1, 1, _H), lambda e: (e, 0, 0)),
            pl.BlockSpec((1, _H, _H), lambda e: (e, 0, 0)),
            pl.BlockSpec((1, 1, _H), lambda e: (e, 0, 0)),
            pl.BlockSpec((1, _H, _A), lambda e: (e, 0, 0)),
            pl.BlockSpec((1, 1, _A), lambda e: (e, 0, 0)),
        ],
        out_specs=pl.BlockSpec((_N, _A), lambda e: (0, 0)),
        out_shape=jax.ShapeDtypeStruct((_N, _A), jnp.float32),
        compiler_params=pltpu.CompilerParams(
            dimension_semantics=("arbitrary",)),
    )(eidx, state, agent_emb, W1, b1.reshape(_E, 1, _H), W2,
      b2.reshape(_E, 1, _H), W3, b3.reshape(_E, 1, _A))


def kernel(state, assigner_logits, agent_emb, W1, b1, W2, b2, W3, b3):
    u = jax.random.uniform(jax.random.key(1), (_B, _G, _E), jnp.float32,
                           1e-6, 1.0 - 1e-6)
    gumbel = -jnp.log(-jnp.log(u))
    scores = assigner_logits[None, :, :] + gumbel
    eidx = jnp.argmax(scores, axis=-1).reshape(_N, 1).astype(jnp.int32)
    out = _run_mlp(eidx, state, agent_emb, W1, b1, W2, b2, W3, b3)
    return out.reshape(_B, _G, _A)
